# baseline (speedup 1.0000x reference)
import functools

import jax
import jax.numpy as jnp
from jax import lax
from jax.experimental import pallas as pl
from jax.experimental.pallas import tpu as pltpu
from jax.experimental.pallas import tpu_sc as plsc

B = 16384
D = 16
NROWS = 1000000
PACK = 8
SUPER = 128 * PACK
PROWS = 125056
RCH = 1024
RGRID = 123
GCHUNK = 128


def _reformat_body(xig, xim, oig, oim):
  for x, o in ((xig, oig), (xim, oim)):
    xt = jnp.transpose(x[...])
    rows = []
    for sb in range(PACK):
      rows.append(jnp.concatenate(
          [xt[1024 * sb + 128 * s:1024 * sb + 128 * s + 128, :]
           for s in range(PACK)], axis=1))
    o[...] = jnp.concatenate(rows, axis=0)


@jax.jit
def _tc_reformat(tig, tim):
  tbl = jax.ShapeDtypeStruct((PROWS, PACK * D), jnp.float32)
  in_spec = pl.BlockSpec((D, PACK * SUPER), lambda c: (0, c))
  out_spec = pl.BlockSpec((RCH, PACK * D), lambda c: (c, 0))
  return pl.pallas_call(
      _reformat_body,
      grid=(RGRID,),
      in_specs=[in_spec] * 2,
      out_specs=[out_spec] * 2,
      out_shape=(tbl, tbl),
  )(tig, tim)


def _sc_gather_body(nc, ns, bpw,
                    uid, iid, tug, tig, tum, tim,
                    oug, oig, oum, oim,
                    xu, xi, bug, big, bum, bim, sem):
  wid = lax.axis_index("s") * nc + lax.axis_index("c")
  base = wid * bpw
  pltpu.sync_copy(uid.at[pl.ds(base, bpw)], xu)
  pltpu.sync_copy(iid.at[pl.ds(base, bpw)], xi)
  for k in range(bpw // D):
    sl = pl.ds(k * D, D)
    v = xi[sl]
    xi[sl] = jnp.bitwise_or(
        lax.shift_left(lax.shift_right_logical(v, 10), 7),
        jnp.bitwise_and(v, 127))
  for k in range(bpw // GCHUNK):
    sl = pl.ds(k * GCHUNK, GCHUNK)
    copies = [
        pltpu.async_copy(tug.at[xu.at[sl]], bug, sem),
        pltpu.async_copy(tig.at[xi.at[sl]], big, sem),
        pltpu.async_copy(tum.at[xu.at[sl]], bum, sem),
        pltpu.async_copy(tim.at[xi.at[sl]], bim, sem),
    ]
    for c in copies:
      c.wait()
    out_sl = pl.ds(base + k * GCHUNK, GCHUNK)
    pltpu.sync_copy(bug, oug.at[out_sl])
    pltpu.sync_copy(big, oig.at[out_sl])
    pltpu.sync_copy(bum, oum.at[out_sl])
    pltpu.sync_copy(bim, oim.at[out_sl])


@jax.jit
def _sc_gather(uid, iid, tug, tig, tum, tim):
  info = plsc.get_sparse_core_info()
  nc, ns = info.num_cores, info.num_subcores
  nw = nc * ns
  bpw = B // nw
  mesh = plsc.VectorSubcoreMesh(core_axis_name="c", subcore_axis_name="s")
  rowo = jax.ShapeDtypeStruct((B, D), jnp.float32)
  packo = jax.ShapeDtypeStruct((B, PACK * D), jnp.float32)
  body = functools.partial(_sc_gather_body, nc, ns, bpw)
  return pl.kernel(
      body,
      mesh=mesh,
      compiler_params=pltpu.CompilerParams(use_tc_tiling_on_sc=False),
      out_type=(rowo, packo, rowo, packo),
      scratch_types=[
          pltpu.VMEM((bpw,), jnp.int32),
          pltpu.VMEM((bpw,), jnp.int32),
          pltpu.VMEM((GCHUNK, D), jnp.float32),
          pltpu.VMEM((GCHUNK, PACK * D), jnp.float32),
          pltpu.VMEM((GCHUNK, D), jnp.float32),
          pltpu.VMEM((GCHUNK, PACK * D), jnp.float32),
          pltpu.SemaphoreType.DMA,
      ],
  )(uid, iid, tug, tig, tum, tim)


HCH = 1024


def _extract(packed, sub):
  acc = jnp.zeros((HCH, D), jnp.float32)
  for s in range(PACK):
    acc = acc + jnp.where(sub == s, packed[:, s * D:(s + 1) * D], 0.0)
  return acc


def _head_body(ugr, pig, umr, pim, iid2, W0, b0, out):
  f32 = jnp.float32
  si = jnp.bitwise_and(lax.shift_right_logical(iid2[...], 7), 7)
  ug = ugr[...]
  um = umr[...]
  ig = _extract(pig[...], si)
  im = _extract(pim[...], si)
  h0 = (jnp.dot(um, W0[0:D, :], preferred_element_type=f32)
        + jnp.dot(im, W0[D:2 * D, :], preferred_element_type=f32)
        + b0[...])
  gmf = ug * ig
  out[...] = jnp.concatenate(
      [h0, gmf, jnp.zeros((HCH, 128 - 64 - D), f32)], axis=1)


@jax.jit
def _tc_head(ugr, pig, umr, pim, iid2, W0, b0):
  rspec = pl.BlockSpec((HCH, D), lambda c: (c, 0))
  bspec = pl.BlockSpec((HCH, PACK * D), lambda c: (c, 0))
  ispec = pl.BlockSpec((HCH, 1), lambda c: (c, 0))
  wspec = pl.BlockSpec((2 * D, 64), lambda c: (0, 0))
  b0spec = pl.BlockSpec((64,), lambda c: (0,))
  return pl.pallas_call(
      _head_body,
      grid=(B // HCH,),
      in_specs=[rspec, bspec, rspec, bspec, ispec, wspec, b0spec],
      out_specs=pl.BlockSpec((HCH, PACK * D), lambda c: (c, 0)),
      out_shape=jax.ShapeDtypeStruct((B, PACK * D), jnp.float32),
  )(ugr, pig, umr, pim, iid2, W0, b0)


def _bn_relu(x, g, be):
  mean = jnp.mean(x, axis=0)
  var = jnp.mean((x - mean) ** 2, axis=0)
  x = (x - mean) * lax.rsqrt(var + 1e-5) * g + be
  return jnp.maximum(x, 0.0)


def _tail_body(hg, g0, be0, W1, b1, g1, be1,
               W2, b2, g2, be2, W3, b3, g3, be3,
               Wp, bp, out):
  f32 = jnp.float32
  x = _bn_relu(hg[:, 0:64], g0[...], be0[...])
  gmf = hg[:, 64:64 + D]
  x = jnp.dot(x, W1[...], preferred_element_type=f32) + b1[...]
  x = _bn_relu(x, g1[...], be1[...])
  x = jnp.dot(x, W2[...], preferred_element_type=f32) + b2[...]
  x = _bn_relu(x, g2[...], be2[...])
  x = jnp.dot(x, W3[...], preferred_element_type=f32) + b3[...]
  x = _bn_relu(x, g3[...], be3[...])
  logit = (jnp.dot(gmf, Wp[0:D, :], preferred_element_type=f32)
           + jnp.dot(x, Wp[D:D + 8, :], preferred_element_type=f32)
           + bp[...])
  out[...] = jax.nn.sigmoid(logit)


@jax.jit
def _tc_tail(hg, *weights):
  return pl.pallas_call(
      _tail_body,
      out_shape=jax.ShapeDtypeStruct((B, 1), jnp.float32),
  )(hg, *weights)


def kernel(user_indices, item_indices, user_gmf, item_gmf, user_mlp, item_mlp,
           W0, b0, g0, be0, W1, b1, g1, be1, W2, b2, g2, be2, W3, b3, g3, be3,
           Wp, bp):
  uid = user_indices.astype(jnp.int32)
  iid = item_indices.astype(jnp.int32)
  rig, rim = _tc_reformat(item_gmf.T, item_mlp.T)
  ugr, pig, umr, pim = _sc_gather(uid, iid, user_gmf, rig, user_mlp, rim)
  hg = _tc_head(ugr, pig, umr, pim, iid.reshape(B, 1), W0, b0)
  pred = _tc_tail(hg, g0, be0, W1, b1, g1, be1,
                  W2, b2, g2, be2, W3, b3, g3, be3, Wp, bp)
  return jnp.squeeze(pred, axis=-1)


# speedup vs baseline: 3.8683x; 3.8683x over previous
import functools

import jax
import jax.numpy as jnp
from jax import lax
from jax.experimental import pallas as pl
from jax.experimental.pallas import tpu as pltpu
from jax.experimental.pallas import tpu_sc as plsc

B = 16384
D = 16
NROWS = 1000000
GRP = 16


def _sc_gather_body(nc, ns, bpw,
                    uid, iid, tug, tig, tum, tim,
                    oug, oig, oum, oim,
                    xu, xi,
                    ba, bb, cug, cig, cum, cim, sem):
  wid = lax.axis_index("s") * nc + lax.axis_index("c")
  base = wid * bpw
  pltpu.sync_copy(uid.at[pl.ds(base, bpw)], xu)
  pltpu.sync_copy(iid.at[pl.ds(base, bpw)], xi)
  riota = lax.iota(jnp.int32, GRP)

  def phase(g, idx_v, t0, t1, out0, out1):
    vec = idx_v[pl.ds(g * GRP, GRP)]
    offs = lax.shift_left(lax.shift_right_logical(vec, 7), 7)
    for r in range(GRP):
      off = pl.multiple_of(offs[r], 128)
      pltpu.async_copy(t0.at[:, pl.ds(off, 128)], ba.at[r], sem)
      pltpu.async_copy(t1.at[:, pl.ds(off, 128)], bb.at[r], sem)
    for buf in (ba, bb):
      for r in range(GRP):
        pltpu.make_async_copy(t0.at[:, pl.ds(0, 128)], buf.at[r], sem).wait()
    lanes = jnp.bitwise_and(idx_v[pl.ds(g * GRP, GRP)], 127)
    for buf, out in ((ba, out0), (bb, out1)):
      for t in range(D):
        tvec = jnp.full((GRP,), t, jnp.int32)
        col = plsc.load_gather(buf, [riota, tvec, lanes])
        plsc.store_scatter(out, [riota, tvec], col)

  def group(g, _):
    phase(g, xu, tug, tum, cug, cum)
    phase(g, xi, tig, tim, cig, cim)
    osl = pl.ds(base + g * GRP, GRP)
    pltpu.sync_copy(cug, oug.at[osl])
    pltpu.sync_copy(cig, oig.at[osl])
    pltpu.sync_copy(cum, oum.at[osl])
    pltpu.sync_copy(cim, oim.at[osl])
    return _

  lax.fori_loop(0, bpw // GRP, group, 0)


@jax.jit
def _sc_gather(uid, iid, tug, tig, tum, tim):
  info = plsc.get_sparse_core_info()
  nc, ns = info.num_cores, info.num_subcores
  nw = nc * ns
  bpw = B // nw
  mesh = plsc.VectorSubcoreMesh(core_axis_name="c", subcore_axis_name="s")
  rowo = jax.ShapeDtypeStruct((B, D), jnp.float32)
  body = functools.partial(_sc_gather_body, nc, ns, bpw)
  return pl.kernel(
      body,
      mesh=mesh,
      compiler_params=pltpu.CompilerParams(needs_layout_passes=False),
      out_type=(rowo, rowo, rowo, rowo),
      scratch_types=[
          pltpu.VMEM((bpw,), jnp.int32),
          pltpu.VMEM((bpw,), jnp.int32),
          pltpu.VMEM((GRP, D, 128), jnp.float32),
          pltpu.VMEM((GRP, D, 128), jnp.float32),
          pltpu.VMEM((GRP, D), jnp.float32),
          pltpu.VMEM((GRP, D), jnp.float32),
          pltpu.VMEM((GRP, D), jnp.float32),
          pltpu.VMEM((GRP, D), jnp.float32),
          pltpu.SemaphoreType.DMA,
      ],
  )(uid, iid, tug, tig, tum, tim)


HCH = 1024


def _head_body(ug, ig, um, im, W0, b0, out):
  f32 = jnp.float32
  h0 = (jnp.dot(um[...], W0[0:D, :], preferred_element_type=f32)
        + jnp.dot(im[...], W0[D:2 * D, :], preferred_element_type=f32)
        + b0[...])
  gmf = ug[...] * ig[...]
  out[...] = jnp.concatenate(
      [h0, gmf, jnp.zeros((HCH, 128 - 64 - D), f32)], axis=1)


@jax.jit
def _tc_head(ug, ig, um, im, W0, b0):
  rspec = pl.BlockSpec((HCH, D), lambda c: (c, 0))
  wspec = pl.BlockSpec((2 * D, 64), lambda c: (0, 0))
  b0spec = pl.BlockSpec((64,), lambda c: (0,))
  return pl.pallas_call(
      _head_body,
      grid=(B // HCH,),
      in_specs=[rspec] * 4 + [wspec, b0spec],
      out_specs=pl.BlockSpec((HCH, 128), lambda c: (c, 0)),
      out_shape=jax.ShapeDtypeStruct((B, 128), jnp.float32),
  )(ug, ig, um, im, W0, b0)


def _bn_relu(x, g, be):
  mean = jnp.mean(x, axis=0)
  var = jnp.mean((x - mean) ** 2, axis=0)
  x = (x - mean) * lax.rsqrt(var + 1e-5) * g + be
  return jnp.maximum(x, 0.0)


def _tail_body(hg, g0, be0, W1, b1, g1, be1,
               W2, b2, g2, be2, W3, b3, g3, be3,
               Wp, bp, out):
  f32 = jnp.float32
  x = _bn_relu(hg[:, 0:64], g0[...], be0[...])
  gmf = hg[:, 64:64 + D]
  x = jnp.dot(x, W1[...], preferred_element_type=f32) + b1[...]
  x = _bn_relu(x, g1[...], be1[...])
  x = jnp.dot(x, W2[...], preferred_element_type=f32) + b2[...]
  x = _bn_relu(x, g2[...], be2[...])
  x = jnp.dot(x, W3[...], preferred_element_type=f32) + b3[...]
  x = _bn_relu(x, g3[...], be3[...])
  logit = (jnp.dot(gmf, Wp[0:D, :], preferred_element_type=f32)
           + jnp.dot(x, Wp[D:D + 8, :], preferred_element_type=f32)
           + bp[...])
  out[...] = jax.nn.sigmoid(logit)


@jax.jit
def _tc_tail(hg, *weights):
  return pl.pallas_call(
      _tail_body,
      out_shape=jax.ShapeDtypeStruct((B, 1), jnp.float32),
  )(hg, *weights)


def kernel(user_indices, item_indices, user_gmf, item_gmf, user_mlp, item_mlp,
           W0, b0, g0, be0, W1, b1, g1, be1, W2, b2, g2, be2, W3, b3, g3, be3,
           Wp, bp):
  uid = user_indices.astype(jnp.int32)
  iid = item_indices.astype(jnp.int32)
  ug, ig, um, im = _sc_gather(uid, iid, user_gmf.T, item_gmf.T,
                              user_mlp.T, item_mlp.T)
  hg = _tc_head(ug, ig, um, im, W0, b0)
  pred = _tc_tail(hg, g0, be0, W1, b1, g1, be1,
                  W2, b2, g2, be2, W3, b3, g3, be3, Wp, bp)
  return jnp.squeeze(pred, axis=-1)


# speedup vs baseline: 3.9392x; 1.0183x over previous
import functools

import jax
import jax.numpy as jnp
from jax import lax
from jax.experimental import pallas as pl
from jax.experimental.pallas import tpu as pltpu
from jax.experimental.pallas import tpu_sc as plsc

B = 16384
D = 16
NROWS = 1000000
GRP = 16


def _sc_gather_body(nc, ns, bpw,
                    uid, iid, tug, tig, tum, tim,
                    oug, oig, oum, oim,
                    xu, xi,
                    ba, bb, cug, cig, cum, cim, sem, wsem):
  wid = lax.axis_index("s") * nc + lax.axis_index("c")
  base = wid * bpw
  pltpu.sync_copy(uid.at[pl.ds(base, bpw)], xu)
  pltpu.sync_copy(iid.at[pl.ds(base, bpw)], xi)
  riota = lax.iota(jnp.int32, GRP)

  def phase(g, idx_v, t0, t1, out0, out1):
    vec = idx_v[pl.ds(g * GRP, GRP)]
    offs = lax.shift_left(lax.shift_right_logical(vec, 7), 7)
    for r in range(GRP):
      off = pl.multiple_of(offs[r], 128)
      pltpu.async_copy(t0.at[:, pl.ds(off, 128)], ba.at[r], sem)
      pltpu.async_copy(t1.at[:, pl.ds(off, 128)], bb.at[r], sem)
    for buf in (ba, bb):
      for r in range(GRP):
        pltpu.make_async_copy(t0.at[:, pl.ds(0, 128)], buf.at[r], sem).wait()
    lanes = jnp.bitwise_and(idx_v[pl.ds(g * GRP, GRP)], 127)
    for buf, out in ((ba, out0), (bb, out1)):
      for t in range(D):
        tvec = jnp.full((GRP,), t, jnp.int32)
        col = plsc.load_gather(buf, [riota, tvec, lanes])
        plsc.store_scatter(out, [riota, tvec], col)

  def group(g, _):
    posl = pl.ds(base + (g - 1) * GRP, GRP)

    @pl.when(g > 0)
    def _drain():
      for c, o in ((cug, oug), (cig, oig), (cum, oum), (cim, oim)):
        pltpu.make_async_copy(c, o.at[posl], wsem).wait()

    phase(g, xu, tug, tum, cug, cum)
    phase(g, xi, tig, tim, cig, cim)
    osl = pl.ds(base + g * GRP, GRP)
    for c, o in ((cug, oug), (cig, oig), (cum, oum), (cim, oim)):
      pltpu.async_copy(c, o.at[osl], wsem)
    return _

  ng = bpw // GRP
  lax.fori_loop(0, ng, group, 0)
  losl = pl.ds(base + (ng - 1) * GRP, GRP)
  for c, o in ((cug, oug), (cig, oig), (cum, oum), (cim, oim)):
    pltpu.make_async_copy(c, o.at[losl], wsem).wait()


@jax.jit
def _sc_gather(uid, iid, tug, tig, tum, tim):
  info = plsc.get_sparse_core_info()
  nc, ns = info.num_cores, info.num_subcores
  nw = nc * ns
  bpw = B // nw
  mesh = plsc.VectorSubcoreMesh(core_axis_name="c", subcore_axis_name="s")
  rowo = jax.ShapeDtypeStruct((B, D), jnp.float32)
  body = functools.partial(_sc_gather_body, nc, ns, bpw)
  return pl.kernel(
      body,
      mesh=mesh,
      compiler_params=pltpu.CompilerParams(needs_layout_passes=False),
      out_type=(rowo, rowo, rowo, rowo),
      scratch_types=[
          pltpu.VMEM((bpw,), jnp.int32),
          pltpu.VMEM((bpw,), jnp.int32),
          pltpu.VMEM((GRP, D, 128), jnp.float32),
          pltpu.VMEM((GRP, D, 128), jnp.float32),
          pltpu.VMEM((GRP, D), jnp.float32),
          pltpu.VMEM((GRP, D), jnp.float32),
          pltpu.VMEM((GRP, D), jnp.float32),
          pltpu.VMEM((GRP, D), jnp.float32),
          pltpu.SemaphoreType.DMA,
          pltpu.SemaphoreType.DMA,
      ],
  )(uid, iid, tug, tig, tum, tim)


HCH = 1024


def _head_body(ug, ig, um, im, W0, b0, out):
  f32 = jnp.float32
  h0 = (jnp.dot(um[...], W0[0:D, :], preferred_element_type=f32)
        + jnp.dot(im[...], W0[D:2 * D, :], preferred_element_type=f32)
        + b0[...])
  gmf = ug[...] * ig[...]
  out[...] = jnp.concatenate(
      [h0, gmf, jnp.zeros((HCH, 128 - 64 - D), f32)], axis=1)


@jax.jit
def _tc_head(ug, ig, um, im, W0, b0):
  rspec = pl.BlockSpec((HCH, D), lambda c: (c, 0))
  wspec = pl.BlockSpec((2 * D, 64), lambda c: (0, 0))
  b0spec = pl.BlockSpec((64,), lambda c: (0,))
  return pl.pallas_call(
      _head_body,
      grid=(B // HCH,),
      in_specs=[rspec] * 4 + [wspec, b0spec],
      out_specs=pl.BlockSpec((HCH, 128), lambda c: (c, 0)),
      out_shape=jax.ShapeDtypeStruct((B, 128), jnp.float32),
  )(ug, ig, um, im, W0, b0)


def _bn_relu(x, g, be):
  mean = jnp.mean(x, axis=0)
  var = jnp.mean((x - mean) ** 2, axis=0)
  x = (x - mean) * lax.rsqrt(var + 1e-5) * g + be
  return jnp.maximum(x, 0.0)


def _tail_body(hg, g0, be0, W1, b1, g1, be1,
               W2, b2, g2, be2, W3, b3, g3, be3,
               Wp, bp, out):
  f32 = jnp.float32
  x = _bn_relu(hg[:, 0:64], g0[...], be0[...])
  gmf = hg[:, 64:64 + D]
  x = jnp.dot(x, W1[...], preferred_element_type=f32) + b1[...]
  x = _bn_relu(x, g1[...], be1[...])
  x = jnp.dot(x, W2[...], preferred_element_type=f32) + b2[...]
  x = _bn_relu(x, g2[...], be2[...])
  x = jnp.dot(x, W3[...], preferred_element_type=f32) + b3[...]
  x = _bn_relu(x, g3[...], be3[...])
  logit = (jnp.dot(gmf, Wp[0:D, :], preferred_element_type=f32)
           + jnp.dot(x, Wp[D:D + 8, :], preferred_element_type=f32)
           + bp[...])
  out[...] = jax.nn.sigmoid(logit)


@jax.jit
def _tc_tail(hg, *weights):
  return pl.pallas_call(
      _tail_body,
      out_shape=jax.ShapeDtypeStruct((B, 1), jnp.float32),
  )(hg, *weights)


def kernel(user_indices, item_indices, user_gmf, item_gmf, user_mlp, item_mlp,
           W0, b0, g0, be0, W1, b1, g1, be1, W2, b2, g2, be2, W3, b3, g3, be3,
           Wp, bp):
  uid = user_indices.astype(jnp.int32)
  iid = item_indices.astype(jnp.int32)
  ug, ig, um, im = _sc_gather(uid, iid, user_gmf.T, item_gmf.T,
                              user_mlp.T, item_mlp.T)
  hg = _tc_head(ug, ig, um, im, W0, b0)
  pred = _tc_tail(hg, g0, be0, W1, b1, g1, be1,
                  W2, b2, g2, be2, W3, b3, g3, be3, Wp, bp)
  return jnp.squeeze(pred, axis=-1)


# speedup vs baseline: 4.0884x; 1.0379x over previous
import functools

import jax
import jax.numpy as jnp
from jax import lax
from jax.experimental import pallas as pl
from jax.experimental.pallas import tpu as pltpu
from jax.experimental.pallas import tpu_sc as plsc

B = 16384
D = 16
NROWS = 1000000
GRP = 16


def _sc_gather_body(nc, ns, bpw,
                    uid, iid, tug, tig, tum, tim,
                    oug, oig, oum, oim,
                    xu, xi,
                    ba, bb, cug, cig, cum, cim, sem, sem2, wsem):
  wid = lax.axis_index("s") * nc + lax.axis_index("c")
  base = wid * bpw
  pltpu.sync_copy(uid.at[pl.ds(base, bpw)], xu)
  pltpu.sync_copy(iid.at[pl.ds(base, bpw)], xi)
  riota = lax.iota(jnp.int32, GRP)
  ng = bpw // GRP
  phases = ((xu, tug, cug), (xu, tum, cum), (xi, tig, cig), (xi, tim, cim))
  bufsem = ((ba, sem), (bb, sem2))

  def fire(g, t):
    idx_v, tbl, _ = phases[t]
    buf, sm = bufsem[t % 2]
    vec = idx_v[pl.ds(g * GRP, GRP)]
    offs = lax.shift_left(lax.shift_right_logical(vec, 7), 7)
    for r in range(GRP):
      off = pl.multiple_of(offs[r], 128)
      pltpu.async_copy(tbl.at[:, pl.ds(off, 128)], buf.at[r], sm)

  def drain_extract(g, t):
    idx_v, tbl, out = phases[t]
    buf, sm = bufsem[t % 2]
    for r in range(GRP):
      pltpu.make_async_copy(tug.at[:, pl.ds(0, 128)], buf.at[r], sm).wait()
    lanes = jnp.bitwise_and(idx_v[pl.ds(g * GRP, GRP)], 127)
    for tt in range(D):
      tvec = jnp.full((GRP,), tt, jnp.int32)
      col = plsc.load_gather(buf, [riota, tvec, lanes])
      plsc.store_scatter(out, [riota, tvec], col)

  fire(0, 0)

  def group(g, _):
    posl = pl.ds(base + (g - 1) * GRP, GRP)

    @pl.when(g > 0)
    def _drain():
      for c, o in ((cug, oug), (cig, oig), (cum, oum), (cim, oim)):
        pltpu.make_async_copy(c, o.at[posl], wsem).wait()

    for t in range(4):
      if t < 3:
        fire(g, t + 1)
      else:
        @pl.when(g < ng - 1)
        def _prefetch():
          fire(g + 1, 0)
      drain_extract(g, t)
    osl = pl.ds(base + g * GRP, GRP)
    for c, o in ((cug, oug), (cig, oig), (cum, oum), (cim, oim)):
      pltpu.async_copy(c, o.at[osl], wsem)
    return _

  lax.fori_loop(0, ng, group, 0)
  losl = pl.ds(base + (ng - 1) * GRP, GRP)
  for c, o in ((cug, oug), (cig, oig), (cum, oum), (cim, oim)):
    pltpu.make_async_copy(c, o.at[losl], wsem).wait()


@jax.jit
def _sc_gather(uid, iid, tug, tig, tum, tim):
  info = plsc.get_sparse_core_info()
  nc, ns = info.num_cores, info.num_subcores
  nw = nc * ns
  bpw = B // nw
  mesh = plsc.VectorSubcoreMesh(core_axis_name="c", subcore_axis_name="s")
  rowo = jax.ShapeDtypeStruct((B, D), jnp.float32)
  body = functools.partial(_sc_gather_body, nc, ns, bpw)
  return pl.kernel(
      body,
      mesh=mesh,
      compiler_params=pltpu.CompilerParams(needs_layout_passes=False),
      out_type=(rowo, rowo, rowo, rowo),
      scratch_types=[
          pltpu.VMEM((bpw,), jnp.int32),
          pltpu.VMEM((bpw,), jnp.int32),
          pltpu.VMEM((GRP, D, 128), jnp.float32),
          pltpu.VMEM((GRP, D, 128), jnp.float32),
          pltpu.VMEM((GRP, D), jnp.float32),
          pltpu.VMEM((GRP, D), jnp.float32),
          pltpu.VMEM((GRP, D), jnp.float32),
          pltpu.VMEM((GRP, D), jnp.float32),
          pltpu.SemaphoreType.DMA,
          pltpu.SemaphoreType.DMA,
          pltpu.SemaphoreType.DMA,
      ],
  )(uid, iid, tug, tig, tum, tim)


HCH = 1024


def _head_body(ug, ig, um, im, W0, b0, out):
  f32 = jnp.float32
  h0 = (jnp.dot(um[...], W0[0:D, :], preferred_element_type=f32)
        + jnp.dot(im[...], W0[D:2 * D, :], preferred_element_type=f32)
        + b0[...])
  gmf = ug[...] * ig[...]
  out[...] = jnp.concatenate(
      [h0, gmf, jnp.zeros((HCH, 128 - 64 - D), f32)], axis=1)


@jax.jit
def _tc_head(ug, ig, um, im, W0, b0):
  rspec = pl.BlockSpec((HCH, D), lambda c: (c, 0))
  wspec = pl.BlockSpec((2 * D, 64), lambda c: (0, 0))
  b0spec = pl.BlockSpec((64,), lambda c: (0,))
  return pl.pallas_call(
      _head_body,
      grid=(B // HCH,),
      in_specs=[rspec] * 4 + [wspec, b0spec],
      out_specs=pl.BlockSpec((HCH, 128), lambda c: (c, 0)),
      out_shape=jax.ShapeDtypeStruct((B, 128), jnp.float32),
  )(ug, ig, um, im, W0, b0)


def _bn_relu(x, g, be):
  mean = jnp.mean(x, axis=0)
  var = jnp.mean((x - mean) ** 2, axis=0)
  x = (x - mean) * lax.rsqrt(var + 1e-5) * g + be
  return jnp.maximum(x, 0.0)


def _tail_body(hg, g0, be0, W1, b1, g1, be1,
               W2, b2, g2, be2, W3, b3, g3, be3,
               Wp, bp, out):
  f32 = jnp.float32
  x = _bn_relu(hg[:, 0:64], g0[...], be0[...])
  gmf = hg[:, 64:64 + D]
  x = jnp.dot(x, W1[...], preferred_element_type=f32) + b1[...]
  x = _bn_relu(x, g1[...], be1[...])
  x = jnp.dot(x, W2[...], preferred_element_type=f32) + b2[...]
  x = _bn_relu(x, g2[...], be2[...])
  x = jnp.dot(x, W3[...], preferred_element_type=f32) + b3[...]
  x = _bn_relu(x, g3[...], be3[...])
  logit = (jnp.dot(gmf, Wp[0:D, :], preferred_element_type=f32)
           + jnp.dot(x, Wp[D:D + 8, :], preferred_element_type=f32)
           + bp[...])
  out[...] = jax.nn.sigmoid(logit)


@jax.jit
def _tc_tail(hg, *weights):
  return pl.pallas_call(
      _tail_body,
      out_shape=jax.ShapeDtypeStruct((B, 1), jnp.float32),
  )(hg, *weights)


def kernel(user_indices, item_indices, user_gmf, item_gmf, user_mlp, item_mlp,
           W0, b0, g0, be0, W1, b1, g1, be1, W2, b2, g2, be2, W3, b3, g3, be3,
           Wp, bp):
  uid = user_indices.astype(jnp.int32)
  iid = item_indices.astype(jnp.int32)
  ug, ig, um, im = _sc_gather(uid, iid, user_gmf.T, item_gmf.T,
                              user_mlp.T, item_mlp.T)
  hg = _tc_head(ug, ig, um, im, W0, b0)
  pred = _tc_tail(hg, g0, be0, W1, b1, g1, be1,
                  W2, b2, g2, be2, W3, b3, g3, be3, Wp, bp)
  return jnp.squeeze(pred, axis=-1)


# speedup vs baseline: 4.2048x; 1.0285x over previous
import functools

import jax
import jax.numpy as jnp
from jax import lax
from jax.experimental import pallas as pl
from jax.experimental.pallas import tpu as pltpu
from jax.experimental.pallas import tpu_sc as plsc

B = 16384
D = 16
NROWS = 1000000
GRP = 16


def _sc_gather_body(nc, ns, bpw,
                    uid, iid, tug, tig, tum, tim,
                    oug, oig, oum, oim,
                    xu, xi,
                    ba, bb, cug, cig, cum, cim, sem, sem2, wsem):
  wid = lax.axis_index("s") * nc + lax.axis_index("c")
  base = wid * bpw
  pltpu.sync_copy(uid.at[pl.ds(base, bpw)], xu)
  pltpu.sync_copy(iid.at[pl.ds(base, bpw)], xi)
  riota = lax.iota(jnp.int32, GRP)
  ng = bpw // GRP
  phases = ((xu, tug, cug), (xu, tum, cum), (xi, tig, cig), (xi, tim, cim))
  bufsem = ((ba, sem), (bb, sem2))

  def fire(g, t):
    idx_v, tbl, _ = phases[t]
    buf, sm = bufsem[t % 2]
    vec = idx_v[pl.ds(g * GRP, GRP)]
    offs = lax.shift_left(lax.shift_right_logical(vec, 7), 7)
    for r in range(GRP):
      off = pl.multiple_of(offs[r], 128)
      pltpu.async_copy(tbl.at[:, pl.ds(off, 128)], buf.at[r], sm)

  def drain_extract(g, t):
    idx_v, tbl, out = phases[t]
    buf, sm = bufsem[t % 2]
    for r in range(GRP):
      pltpu.make_async_copy(tug.at[:, pl.ds(0, 128)], buf.at[r], sm).wait()
    lanes = jnp.bitwise_and(idx_v[pl.ds(g * GRP, GRP)], 127)
    for tt in range(D):
      tvec = jnp.full((GRP,), tt, jnp.int32)
      col = plsc.load_gather(buf, [riota, tvec, lanes])
      plsc.store_scatter(out, [riota, tvec], col)

  fire(0, 0)

  def group(g, _):
    posl = pl.ds(base + (g - 1) * GRP, GRP)

    @pl.when(g > 0)
    def _drain():
      for c, o in ((cug, oug), (cig, oig), (cum, oum), (cim, oim)):
        pltpu.make_async_copy(c, o.at[posl], wsem).wait()

    for t in range(4):
      if t < 3:
        fire(g, t + 1)
      else:
        @pl.when(g < ng - 1)
        def _prefetch():
          fire(g + 1, 0)
      drain_extract(g, t)
    osl = pl.ds(base + g * GRP, GRP)
    for c, o in ((cug, oug), (cig, oig), (cum, oum), (cim, oim)):
      pltpu.async_copy(c, o.at[osl], wsem)
    return _

  lax.fori_loop(0, ng, group, 0)
  losl = pl.ds(base + (ng - 1) * GRP, GRP)
  for c, o in ((cug, oug), (cig, oig), (cum, oum), (cim, oim)):
    pltpu.make_async_copy(c, o.at[losl], wsem).wait()


@jax.jit
def _sc_gather(uid, iid, tug, tig, tum, tim):
  info = plsc.get_sparse_core_info()
  nc, ns = info.num_cores, info.num_subcores
  nw = nc * ns
  bpw = B // nw
  mesh = plsc.VectorSubcoreMesh(core_axis_name="c", subcore_axis_name="s")
  rowo = jax.ShapeDtypeStruct((B, D), jnp.float32)
  body = functools.partial(_sc_gather_body, nc, ns, bpw)
  return pl.kernel(
      body,
      mesh=mesh,
      compiler_params=pltpu.CompilerParams(needs_layout_passes=False),
      out_type=(rowo, rowo, rowo, rowo),
      scratch_types=[
          pltpu.VMEM((bpw,), jnp.int32),
          pltpu.VMEM((bpw,), jnp.int32),
          pltpu.VMEM((GRP, D, 128), jnp.float32),
          pltpu.VMEM((GRP, D, 128), jnp.float32),
          pltpu.VMEM((GRP, D), jnp.float32),
          pltpu.VMEM((GRP, D), jnp.float32),
          pltpu.VMEM((GRP, D), jnp.float32),
          pltpu.VMEM((GRP, D), jnp.float32),
          pltpu.SemaphoreType.DMA,
          pltpu.SemaphoreType.DMA,
          pltpu.SemaphoreType.DMA,
      ],
  )(uid, iid, tug, tig, tum, tim)


def _bn_relu(x, g, be):
  mean = jnp.mean(x, axis=0)
  var = jnp.mean((x - mean) ** 2, axis=0)
  x = (x - mean) * lax.rsqrt(var + 1e-5) * g + be
  return jnp.maximum(x, 0.0)


def _tc_body(ug, ig, um, im,
             W0, b0, g0, be0, W1, b1, g1, be1,
             W2, b2, g2, be2, W3, b3, g3, be3,
             Wp, bp, out):
  f32 = jnp.float32
  x = (jnp.dot(um[...], W0[0:D, :], preferred_element_type=f32)
       + jnp.dot(im[...], W0[D:2 * D, :], preferred_element_type=f32)
       + b0[...])
  x = _bn_relu(x, g0[...], be0[...])
  x = jnp.dot(x, W1[...], preferred_element_type=f32) + b1[...]
  x = _bn_relu(x, g1[...], be1[...])
  x = jnp.dot(x, W2[...], preferred_element_type=f32) + b2[...]
  x = _bn_relu(x, g2[...], be2[...])
  x = jnp.dot(x, W3[...], preferred_element_type=f32) + b3[...]
  x = _bn_relu(x, g3[...], be3[...])
  gmf = ug[...] * ig[...]
  logit = (jnp.dot(gmf, Wp[0:D, :], preferred_element_type=f32)
           + jnp.dot(x, Wp[D:D + 8, :], preferred_element_type=f32)
           + bp[...])
  out[...] = jax.nn.sigmoid(logit)


@jax.jit
def _tc_mlp(ug, ig, um, im, *weights):
  return pl.pallas_call(
      _tc_body,
      out_shape=jax.ShapeDtypeStruct((B, 1), jnp.float32),
  )(ug, ig, um, im, *weights)


def kernel(user_indices, item_indices, user_gmf, item_gmf, user_mlp, item_mlp,
           W0, b0, g0, be0, W1, b1, g1, be1, W2, b2, g2, be2, W3, b3, g3, be3,
           Wp, bp):
  uid = user_indices.astype(jnp.int32)
  iid = item_indices.astype(jnp.int32)
  ug, ig, um, im = _sc_gather(uid, iid, user_gmf.T, item_gmf.T,
                              user_mlp.T, item_mlp.T)
  pred = _tc_mlp(ug, ig, um, im,
                 W0, b0, g0, be0, W1, b1, g1, be1,
                 W2, b2, g2, be2, W3, b3, g3, be3, Wp, bp)
  return jnp.squeeze(pred, axis=-1)


# speedup vs baseline: 4.3390x; 1.0319x over previous
import functools

import jax
import jax.numpy as jnp
from jax import lax
from jax.experimental import pallas as pl
from jax.experimental.pallas import tpu as pltpu
from jax.experimental.pallas import tpu_sc as plsc

B = 16384
D = 16
NROWS = 1000000
GRP = 16


def _sc_gather_body(nc, ns, bpw,
                    uid, iid, tug, tig, tum, tim,
                    oug, oig, oum, oim,
                    xu, xi,
                    ba, bb, cug, cig, cum, cim, sem, sem2, sem3, sem4, wsem):
  wid = lax.axis_index("s") * nc + lax.axis_index("c")
  base = wid * bpw
  pltpu.sync_copy(uid.at[pl.ds(base, bpw)], xu)
  pltpu.sync_copy(iid.at[pl.ds(base, bpw)], xi)
  riota = lax.iota(jnp.int32, GRP)
  ng = bpw // GRP
  phases = ((xu, tug, cug), (xu, tum, cum), (xi, tig, cig), (xi, tim, cim))
  bufsem = ((ba, sem, sem2), (bb, sem3, sem4))

  def fire(g, t):
    idx_v, tbl, _ = phases[t]
    buf, sma, smb = bufsem[t % 2]
    vec = idx_v[pl.ds(g * GRP, GRP)]
    offs = lax.shift_left(lax.shift_right_logical(vec, 7), 7)
    for r in range(GRP):
      off = pl.multiple_of(offs[r], 128)
      pltpu.async_copy(tbl.at[:, pl.ds(off, 128)], buf.at[r],
                       sma if r % 2 == 0 else smb)

  def drain_extract(g, t):
    idx_v, tbl, out = phases[t]
    buf, sma, smb = bufsem[t % 2]
    for r in range(GRP):
      pltpu.make_async_copy(tug.at[:, pl.ds(0, 128)], buf.at[r],
                            sma if r % 2 == 0 else smb).wait()
    lanes = jnp.bitwise_and(idx_v[pl.ds(g * GRP, GRP)], 127)
    for tt in range(D):
      tvec = jnp.full((GRP,), tt, jnp.int32)
      col = plsc.load_gather(buf, [riota, tvec, lanes])
      plsc.store_scatter(out, [riota, tvec], col)

  fire(0, 0)

  def group(g, _):
    posl = pl.ds(base + (g - 1) * GRP, GRP)

    @pl.when(g > 0)
    def _drain():
      for c, o in ((cug, oug), (cig, oig), (cum, oum), (cim, oim)):
        pltpu.make_async_copy(c, o.at[posl], wsem).wait()

    for t in range(4):
      if t < 3:
        fire(g, t + 1)
      else:
        @pl.when(g < ng - 1)
        def _prefetch():
          fire(g + 1, 0)
      drain_extract(g, t)
    osl = pl.ds(base + g * GRP, GRP)
    for c, o in ((cug, oug), (cig, oig), (cum, oum), (cim, oim)):
      pltpu.async_copy(c, o.at[osl], wsem)
    return _

  lax.fori_loop(0, ng, group, 0)
  losl = pl.ds(base + (ng - 1) * GRP, GRP)
  for c, o in ((cug, oug), (cig, oig), (cum, oum), (cim, oim)):
    pltpu.make_async_copy(c, o.at[losl], wsem).wait()


@jax.jit
def _sc_gather(uid, iid, tug, tig, tum, tim):
  info = plsc.get_sparse_core_info()
  nc, ns = info.num_cores, info.num_subcores
  nw = nc * ns
  bpw = B // nw
  mesh = plsc.VectorSubcoreMesh(core_axis_name="c", subcore_axis_name="s")
  rowo = jax.ShapeDtypeStruct((B, D), jnp.float32)
  body = functools.partial(_sc_gather_body, nc, ns, bpw)
  return pl.kernel(
      body,
      mesh=mesh,
      compiler_params=pltpu.CompilerParams(needs_layout_passes=False),
      out_type=(rowo, rowo, rowo, rowo),
      scratch_types=[
          pltpu.VMEM((bpw,), jnp.int32),
          pltpu.VMEM((bpw,), jnp.int32),
          pltpu.VMEM((GRP, D, 128), jnp.float32),
          pltpu.VMEM((GRP, D, 128), jnp.float32),
          pltpu.VMEM((GRP, D), jnp.float32),
          pltpu.VMEM((GRP, D), jnp.float32),
          pltpu.VMEM((GRP, D), jnp.float32),
          pltpu.VMEM((GRP, D), jnp.float32),
          pltpu.SemaphoreType.DMA,
          pltpu.SemaphoreType.DMA,
          pltpu.SemaphoreType.DMA,
          pltpu.SemaphoreType.DMA,
          pltpu.SemaphoreType.DMA,
      ],
  )(uid, iid, tug, tig, tum, tim)


def _bn_relu(x, g, be):
  mean = jnp.mean(x, axis=0)
  var = jnp.mean((x - mean) ** 2, axis=0)
  x = (x - mean) * lax.rsqrt(var + 1e-5) * g + be
  return jnp.maximum(x, 0.0)


def _tc_body(ug, ig, um, im,
             W0, b0, g0, be0, W1, b1, g1, be1,
             W2, b2, g2, be2, W3, b3, g3, be3,
             Wp, bp, out):
  f32 = jnp.float32
  x = (jnp.dot(um[...], W0[0:D, :], preferred_element_type=f32)
       + jnp.dot(im[...], W0[D:2 * D, :], preferred_element_type=f32)
       + b0[...])
  x = _bn_relu(x, g0[...], be0[...])
  x = jnp.dot(x, W1[...], preferred_element_type=f32) + b1[...]
  x = _bn_relu(x, g1[...], be1[...])
  x = jnp.dot(x, W2[...], preferred_element_type=f32) + b2[...]
  x = _bn_relu(x, g2[...], be2[...])
  x = jnp.dot(x, W3[...], preferred_element_type=f32) + b3[...]
  x = _bn_relu(x, g3[...], be3[...])
  gmf = ug[...] * ig[...]
  logit = (jnp.dot(gmf, Wp[0:D, :], preferred_element_type=f32)
           + jnp.dot(x, Wp[D:D + 8, :], preferred_element_type=f32)
           + bp[...])
  out[...] = jax.nn.sigmoid(logit)


@jax.jit
def _tc_mlp(ug, ig, um, im, *weights):
  return pl.pallas_call(
      _tc_body,
      out_shape=jax.ShapeDtypeStruct((B, 1), jnp.float32),
  )(ug, ig, um, im, *weights)


def kernel(user_indices, item_indices, user_gmf, item_gmf, user_mlp, item_mlp,
           W0, b0, g0, be0, W1, b1, g1, be1, W2, b2, g2, be2, W3, b3, g3, be3,
           Wp, bp):
  uid = user_indices.astype(jnp.int32)
  iid = item_indices.astype(jnp.int32)
  ug, ig, um, im = _sc_gather(uid, iid, user_gmf.T, item_gmf.T,
                              user_mlp.T, item_mlp.T)
  pred = _tc_mlp(ug, ig, um, im,
                 W0, b0, g0, be0, W1, b1, g1, be1,
                 W2, b2, g2, be2, W3, b3, g3, be3, Wp, bp)
  return jnp.squeeze(pred, axis=-1)


# speedup vs baseline: 4.4809x; 1.0327x over previous
import functools

import jax
import jax.numpy as jnp
from jax import lax
from jax.experimental import pallas as pl
from jax.experimental.pallas import tpu as pltpu
from jax.experimental.pallas import tpu_sc as plsc

B = 16384
D = 16
NROWS = 1000000
GRP = 16


def _sc_gather_body(nc, ns, bpw,
                    uid, iid, tug, tig, tum, tim,
                    oug, oig, oum, oim,
                    xu, xi,
                    ba, bb, cug, cig, cum, cim,
                    sem, sem2, sem3, sem4, sem5, sem6, sem7, sem8, wsem):
  wid = lax.axis_index("s") * nc + lax.axis_index("c")
  base = wid * bpw
  pltpu.sync_copy(uid.at[pl.ds(base, bpw)], xu)
  pltpu.sync_copy(iid.at[pl.ds(base, bpw)], xi)
  riota = lax.iota(jnp.int32, GRP)
  ng = bpw // GRP
  phases = ((xu, tug, cug), (xu, tum, cum), (xi, tig, cig), (xi, tim, cim))
  bufsem = ((ba, (sem, sem2, sem5, sem6)), (bb, (sem3, sem4, sem7, sem8)))

  def fire(g, t):
    idx_v, tbl, _ = phases[t]
    buf, sms = bufsem[t % 2]
    vec = idx_v[pl.ds(g * GRP, GRP)]
    offs = lax.shift_left(lax.shift_right_logical(vec, 7), 7)
    for r in range(GRP):
      off = pl.multiple_of(offs[r], 128)
      pltpu.async_copy(tbl.at[:, pl.ds(off, 128)], buf.at[r], sms[r % 4])

  def drain_extract(g, t):
    idx_v, tbl, out = phases[t]
    buf, sms = bufsem[t % 2]
    for r in range(GRP):
      pltpu.make_async_copy(tug.at[:, pl.ds(0, 128)], buf.at[r],
                            sms[r % 4]).wait()
    lanes = jnp.bitwise_and(idx_v[pl.ds(g * GRP, GRP)], 127)
    for tt in range(D):
      tvec = jnp.full((GRP,), tt, jnp.int32)
      col = plsc.load_gather(buf, [riota, tvec, lanes])
      plsc.store_scatter(out, [riota, tvec], col)

  fire(0, 0)

  def group(g, _):
    posl = pl.ds(base + (g - 1) * GRP, GRP)

    @pl.when(g > 0)
    def _drain():
      for c, o in ((cug, oug), (cig, oig), (cum, oum), (cim, oim)):
        pltpu.make_async_copy(c, o.at[posl], wsem).wait()

    for t in range(4):
      if t < 3:
        fire(g, t + 1)
      else:
        @pl.when(g < ng - 1)
        def _prefetch():
          fire(g + 1, 0)
      drain_extract(g, t)
    osl = pl.ds(base + g * GRP, GRP)
    for c, o in ((cug, oug), (cig, oig), (cum, oum), (cim, oim)):
      pltpu.async_copy(c, o.at[osl], wsem)
    return _

  lax.fori_loop(0, ng, group, 0)
  losl = pl.ds(base + (ng - 1) * GRP, GRP)
  for c, o in ((cug, oug), (cig, oig), (cum, oum), (cim, oim)):
    pltpu.make_async_copy(c, o.at[losl], wsem).wait()


@jax.jit
def _sc_gather(uid, iid, tug, tig, tum, tim):
  info = plsc.get_sparse_core_info()
  nc, ns = info.num_cores, info.num_subcores
  nw = nc * ns
  bpw = B // nw
  mesh = plsc.VectorSubcoreMesh(core_axis_name="c", subcore_axis_name="s")
  rowo = jax.ShapeDtypeStruct((B, D), jnp.float32)
  body = functools.partial(_sc_gather_body, nc, ns, bpw)
  return pl.kernel(
      body,
      mesh=mesh,
      compiler_params=pltpu.CompilerParams(needs_layout_passes=False),
      out_type=(rowo, rowo, rowo, rowo),
      scratch_types=[
          pltpu.VMEM((bpw,), jnp.int32),
          pltpu.VMEM((bpw,), jnp.int32),
          pltpu.VMEM((GRP, D, 128), jnp.float32),
          pltpu.VMEM((GRP, D, 128), jnp.float32),
          pltpu.VMEM((GRP, D), jnp.float32),
          pltpu.VMEM((GRP, D), jnp.float32),
          pltpu.VMEM((GRP, D), jnp.float32),
          pltpu.VMEM((GRP, D), jnp.float32),
          pltpu.SemaphoreType.DMA,
          pltpu.SemaphoreType.DMA,
          pltpu.SemaphoreType.DMA,
          pltpu.SemaphoreType.DMA,
          pltpu.SemaphoreType.DMA,
          pltpu.SemaphoreType.DMA,
          pltpu.SemaphoreType.DMA,
          pltpu.SemaphoreType.DMA,
          pltpu.SemaphoreType.DMA,
      ],
  )(uid, iid, tug, tig, tum, tim)


def _bn_relu(x, g, be):
  mean = jnp.mean(x, axis=0)
  var = jnp.mean((x - mean) ** 2, axis=0)
  x = (x - mean) * lax.rsqrt(var + 1e-5) * g + be
  return jnp.maximum(x, 0.0)


def _tc_body(ug, ig, um, im,
             W0, b0, g0, be0, W1, b1, g1, be1,
             W2, b2, g2, be2, W3, b3, g3, be3,
             Wp, bp, out):
  f32 = jnp.float32
  x = (jnp.dot(um[...], W0[0:D, :], preferred_element_type=f32)
       + jnp.dot(im[...], W0[D:2 * D, :], preferred_element_type=f32)
       + b0[...])
  x = _bn_relu(x, g0[...], be0[...])
  x = jnp.dot(x, W1[...], preferred_element_type=f32) + b1[...]
  x = _bn_relu(x, g1[...], be1[...])
  x = jnp.dot(x, W2[...], preferred_element_type=f32) + b2[...]
  x = _bn_relu(x, g2[...], be2[...])
  x = jnp.dot(x, W3[...], preferred_element_type=f32) + b3[...]
  x = _bn_relu(x, g3[...], be3[...])
  gmf = ug[...] * ig[...]
  logit = (jnp.dot(gmf, Wp[0:D, :], preferred_element_type=f32)
           + jnp.dot(x, Wp[D:D + 8, :], preferred_element_type=f32)
           + bp[...])
  out[...] = jax.nn.sigmoid(logit)


@jax.jit
def _tc_mlp(ug, ig, um, im, *weights):
  return pl.pallas_call(
      _tc_body,
      out_shape=jax.ShapeDtypeStruct((B, 1), jnp.float32),
  )(ug, ig, um, im, *weights)


def kernel(user_indices, item_indices, user_gmf, item_gmf, user_mlp, item_mlp,
           W0, b0, g0, be0, W1, b1, g1, be1, W2, b2, g2, be2, W3, b3, g3, be3,
           Wp, bp):
  uid = user_indices.astype(jnp.int32)
  iid = item_indices.astype(jnp.int32)
  ug, ig, um, im = _sc_gather(uid, iid, user_gmf.T, item_gmf.T,
                              user_mlp.T, item_mlp.T)
  pred = _tc_mlp(ug, ig, um, im,
                 W0, b0, g0, be0, W1, b1, g1, be1,
                 W2, b2, g2, be2, W3, b3, g3, be3, Wp, bp)
  return jnp.squeeze(pred, axis=-1)
